# baseline (speedup 1.0000x reference)
import math

import jax
import jax.numpy as jnp
from jax import lax
from jax.experimental import pallas as pl
from jax.experimental.pallas import tpu as pltpu
from jax.experimental.pallas import tpu_sc as plsc

D_MODEL = 64
SCALE = math.sqrt(D_MODEL)

NC = 2
NS = 16
NW = NC * NS

L = 16
BLK = 128
T_DIM = 200
B_DIM = 4096
NBLK = T_DIM * B_DIM // BLK
BPW = NBLK // NW
BH = B_DIM // BLK
KV = BLK // L
VOC_PAIR = 500000


def _emb_body(idx_hbm, tab_hbm, out_hbm,
              idx_all, rows_v, trs_v, sem_g, sem_w):
    c = lax.axis_index("c")
    s = lax.axis_index("s")
    wid = s * NC + c
    g0 = wid * BPW

    pltpu.sync_copy(idx_hbm.at[pl.ds(g0, BPW)], idx_all)

    def gather(gl, slot):
        return pltpu.make_async_copy(
            tab_hbm.at[idx_all.at[gl]], rows_v.at[slot], sem_g)

    def out_writes(gl, slot):
        g = g0 + gl
        t = g // BH
        bb = g - t * BH
        return [
            pltpu.make_async_copy(
                trs_v.at[slot, pl.ds(dh * 8, 8), pl.ds(0, BLK)],
                out_hbm.at[t, dh, bb], sem_w)
            for dh in range(8)
        ]

    def transpose(slot):
        rows = rows_v.at[slot]
        tp = trs_v.at[slot]
        himask = jnp.full((L,), -65536, jnp.int32)

        @plsc.parallel_loop(0, BLK, unroll=2)
        def _b(b):
            bvec = jnp.full((L,), b, jnp.int32)
            for dblk in range(D_MODEL // (2 * L)):
                v32 = rows[b, pl.ds(dblk * 2 * L, 2 * L)]
                vi = plsc.bitcast(v32, jnp.int32)
                lo = plsc.bitcast(lax.shift_left(vi, 16), jnp.float32)
                hi = plsc.bitcast(lax.bitwise_and(vi, himask), jnp.float32)
                deven = dblk * 2 * L + 2 * lax.iota(jnp.int32, L)
                plsc.store_scatter(tp, [deven, bvec], lo * SCALE)
                plsc.store_scatter(tp, [deven + 1, bvec], hi * SCALE)

    def half(gl, slot, nslot):
        @pl.when(gl + 1 < BPW)
        def _():
            gather(gl + 1, nslot).start()

        gather(gl, slot).wait()

        @pl.when(gl >= 2)
        def _():
            for cp in out_writes(gl - 2, slot):
                cp.wait()

        transpose(slot)
        for cp in out_writes(gl, slot):
            cp.start()

    gather(0, 0).start()

    def loop_body(i, carry):
        half(2 * i, 0, 1)
        half(2 * i + 1, 1, 0)
        return carry

    lax.fori_loop(0, BPW // 2, loop_body, 0)

    for cp in out_writes(BPW - 2, 0):
        cp.wait()
    for cp in out_writes(BPW - 1, 1):
        cp.wait()


@jax.jit
def _emb(xf2, tab2):
    mesh = plsc.VectorSubcoreMesh(
        core_axis_name="c", subcore_axis_name="s",
        num_cores=NC, num_subcores=NS)
    f = pl.kernel(
        _emb_body,
        out_type=jax.ShapeDtypeStruct((T_DIM, 8, BH, 8, BLK), jnp.float32),
        mesh=mesh,
        scratch_types=[
            pltpu.VMEM((BPW, BLK), jnp.int32),
            pltpu.VMEM((2, BLK, D_MODEL), jnp.bfloat16),
            pltpu.VMEM((2, D_MODEL, BLK + 1), jnp.float32),
            pltpu.SemaphoreType.DMA,
            pltpu.SemaphoreType.DMA,
        ],
        compiler_params=pltpu.CompilerParams(
            use_tc_tiling_on_sc=False, needs_layout_passes=False),
    )
    return f(xf2, tab2)


def kernel(x, table):
    b, t = x.shape
    assert (b, t) == (B_DIM, T_DIM)
    xf2 = x.T.reshape(NBLK, BLK).astype(jnp.int32)
    tabh = table.astype(jnp.bfloat16)
    out5 = _emb(xf2, tabh)
    out = out5.transpose(2, 4, 0, 1, 3).reshape(B_DIM, T_DIM, D_MODEL)
    return out


# speedup vs baseline: 1.2586x; 1.2586x over previous
import math

import jax
import jax.numpy as jnp
from jax import lax
from jax.experimental import pallas as pl
from jax.experimental.pallas import tpu as pltpu
from jax.experimental.pallas import tpu_sc as plsc

D_MODEL = 64
SCALE = math.sqrt(D_MODEL)

NC = 2
NS = 16
NW = NC * NS

L = 16
BLK = 128
T_DIM = 200
B_DIM = 4096
NBLK = T_DIM * B_DIM // BLK
BPW = NBLK // NW
BH = B_DIM // BLK
KV = BLK // L
VOC_PAIR = 500000


def _emb_body(idx_hbm, tab_hbm, out_hbm,
              idx_all, rows_v, trs_v, sem_g, sem_w):
    c = lax.axis_index("c")
    s = lax.axis_index("s")
    wid = s * NC + c
    g0 = wid * BPW

    pltpu.sync_copy(idx_hbm.at[pl.ds(g0, BPW)], idx_all)

    def gather(gl, slot):
        return pltpu.make_async_copy(
            tab_hbm.at[idx_all.at[gl]], rows_v.at[slot], sem_g)

    def out_writes(gl, slot):
        g = g0 + gl
        t = g // BH
        bb = g - t * BH
        return [
            pltpu.make_async_copy(
                trs_v.at[slot, pl.ds(dh * 8, 8), pl.ds(0, BLK)],
                out_hbm.at[t, dh, bb], sem_w)
            for dh in range(8)
        ]

    def transpose(slot):
        rows = rows_v.at[slot]
        tp = trs_v.at[slot]

        @plsc.parallel_loop(0, BLK, unroll=2)
        def _b(b):
            bvec = jnp.full((L,), b, jnp.int32)
            for dblk in range(D_MODEL // L):
                didx = dblk * L + lax.iota(jnp.int32, L)
                v = rows[b, pl.ds(dblk * L, L)]
                plsc.store_scatter(tp, [didx, bvec], v * SCALE)

    def half(gl, slot, nslot):
        @pl.when(gl + 1 < BPW)
        def _():
            gather(gl + 1, nslot).start()

        gather(gl, slot).wait()

        @pl.when(gl >= 2)
        def _():
            for cp in out_writes(gl - 2, slot):
                cp.wait()

        transpose(slot)
        for cp in out_writes(gl, slot):
            cp.start()

    gather(0, 0).start()

    def loop_body(i, carry):
        half(2 * i, 0, 1)
        half(2 * i + 1, 1, 0)
        return carry

    lax.fori_loop(0, BPW // 2, loop_body, 0)

    for cp in out_writes(BPW - 2, 0):
        cp.wait()
    for cp in out_writes(BPW - 1, 1):
        cp.wait()


@jax.jit
def _emb(xf2, tab2):
    mesh = plsc.VectorSubcoreMesh(
        core_axis_name="c", subcore_axis_name="s",
        num_cores=NC, num_subcores=NS)
    f = pl.kernel(
        _emb_body,
        out_type=jax.ShapeDtypeStruct((T_DIM, 8, BH, 8, BLK), jnp.float32),
        mesh=mesh,
        scratch_types=[
            pltpu.VMEM((BPW, BLK), jnp.int32),
            pltpu.VMEM((2, BLK, D_MODEL), jnp.float32),
            pltpu.VMEM((2, D_MODEL, BLK + 1), jnp.float32),
            pltpu.SemaphoreType.DMA,
            pltpu.SemaphoreType.DMA,
        ],
        compiler_params=pltpu.CompilerParams(
            use_tc_tiling_on_sc=False, needs_layout_passes=False),
    )
    return f(xf2, tab2)


def kernel(x, table):
    b, t = x.shape
    assert (b, t) == (B_DIM, T_DIM)
    xf2 = x.T.reshape(NBLK, BLK).astype(jnp.int32)
    out5 = _emb(xf2, table)
    out = out5.transpose(2, 4, 0, 1, 3).reshape(B_DIM, T_DIM, D_MODEL)
    return out


# speedup vs baseline: 1.2608x; 1.0018x over previous
import math

import jax
import jax.numpy as jnp
from jax import lax
from jax.experimental import pallas as pl
from jax.experimental.pallas import tpu as pltpu
from jax.experimental.pallas import tpu_sc as plsc

D_MODEL = 64
SCALE = math.sqrt(D_MODEL)

NC = 2
NS = 16
NW = NC * NS

L = 16
BLK = 128
T_DIM = 200
B_DIM = 4096
NBLK = T_DIM * B_DIM // BLK
BPW = NBLK // NW
BH = B_DIM // BLK


def _emb_body(idx_hbm, tab_hbm, out_hbm,
              idx_all, rows_v, trs_v, sem_g, sem_w):
    c = lax.axis_index("c")
    s = lax.axis_index("s")
    wid = s * NC + c
    g0 = wid * BPW

    pltpu.sync_copy(idx_hbm.at[pl.ds(g0, BPW)], idx_all)

    def gather(gl, slot):
        return pltpu.make_async_copy(
            tab_hbm.at[idx_all.at[gl]], rows_v.at[slot], sem_g)

    def out_writes(gl, slot):
        g = g0 + gl
        t = g // BH
        bb = g - t * BH
        return [
            pltpu.make_async_copy(
                trs_v.at[slot, pl.ds(dh * 8, 8), pl.ds(0, BLK)],
                out_hbm.at[t, dh, bb], sem_w)
            for dh in range(8)
        ]

    def transpose(slot):
        rows = rows_v.at[slot]
        tp = trs_v.at[slot]

        @plsc.parallel_loop(0, BLK, unroll=4)
        def _b(b):
            bvec = jnp.full((L,), b, jnp.int32)
            for dblk in range(D_MODEL // L):
                didx = dblk * L + lax.iota(jnp.int32, L)
                v = rows[b, pl.ds(dblk * L, L)]
                plsc.store_scatter(tp, [didx, bvec], v * SCALE)

    def half(gl, slot, nslot):
        @pl.when(gl + 1 < BPW)
        def _():
            gather(gl + 1, nslot).start()

        gather(gl, slot).wait()

        @pl.when(gl >= 2)
        def _():
            for cp in out_writes(gl - 2, slot):
                cp.wait()

        transpose(slot)
        for cp in out_writes(gl, slot):
            cp.start()

    gather(0, 0).start()

    def loop_body(i, carry):
        half(2 * i, 0, 1)
        half(2 * i + 1, 1, 0)
        return carry

    lax.fori_loop(0, BPW // 2, loop_body, 0)

    for cp in out_writes(BPW - 2, 0):
        cp.wait()
    for cp in out_writes(BPW - 1, 1):
        cp.wait()


@jax.jit
def _emb(xf2, tab2):
    mesh = plsc.VectorSubcoreMesh(
        core_axis_name="c", subcore_axis_name="s",
        num_cores=NC, num_subcores=NS)
    f = pl.kernel(
        _emb_body,
        out_type=jax.ShapeDtypeStruct((T_DIM, 8, BH, 8, BLK), jnp.float32),
        mesh=mesh,
        scratch_types=[
            pltpu.VMEM((BPW, BLK), jnp.int32),
            pltpu.VMEM((2, BLK, D_MODEL), jnp.float32),
            pltpu.VMEM((2, D_MODEL, BLK + 1), jnp.float32),
            pltpu.SemaphoreType.DMA,
            pltpu.SemaphoreType.DMA,
        ],
        compiler_params=pltpu.CompilerParams(
            use_tc_tiling_on_sc=False, needs_layout_passes=False),
    )
    return f(xf2, tab2)


def kernel(x, table):
    b, t = x.shape
    assert (b, t) == (B_DIM, T_DIM)
    xf2 = x.T.reshape(NBLK, BLK).astype(jnp.int32)
    out5 = _emb(xf2, table)
    out = out5.transpose(2, 4, 0, 1, 3).reshape(B_DIM, T_DIM, D_MODEL)
    return out
